# baseline (speedup 1.0000x reference)
import jax
import jax.numpy as jnp
from jax.experimental import pallas as pl
from jax.experimental.pallas import tpu as pltpu

_B = 16
_MAX_T = 4096
_S_DIM = 64
_HIDDEN = 128
_NA = 4
_CT = 1024
_NC = _MAX_T // _CT


def _traj_kernel(lens_ref, s_ref, a_ref, w1_ref, b1_ref, w2t_ref, b2_ref,
                 out_ref):
    i = pl.program_id(0)
    length = lens_ref[i]

    @pl.when(i == 0)
    def _init():
        out_ref[0, 0] = 0.0

    acts = a_ref[0, pl.ds(i, 1), :]

    for c in range(_NC):
        @pl.when(c * _CT < length)
        def _chunk(c=c):
            x = s_ref[0, pl.ds(c * _CT, _CT), :]
            h = jnp.tanh(
                jax.lax.dot_general(x, w1_ref[...], (((1,), (0,)), ((), ())),
                                    preferred_element_type=jnp.float32)
                + b1_ref[0])
            zt = jax.lax.dot_general(w2t_ref[...], h, (((1,), (1,)), ((), ())),
                                     preferred_element_type=jnp.float32)
            zt = zt + b2_ref[...]
            m = jnp.max(zt, axis=0, keepdims=True)
            lse = m + jnp.log(jnp.sum(jnp.exp(zt - m), axis=0, keepdims=True))
            logp = zt - lse
            a_c = acts[:, c * _CT:(c + 1) * _CT]
            onehot = (a_c ==
                      jax.lax.broadcasted_iota(jnp.int32, (_NA, _CT), 0))
            t = c * _CT + jax.lax.broadcasted_iota(jnp.int32, (1, _CT), 1)
            sel = jnp.where(onehot & (t < length), logp, 0.0)
            out_ref[0, 0] += jnp.sum(sel)


def kernel(s_i_batch, actions_batch, lengths, W1, b1, W2, b2):
    w2t = W2[:, :_NA].T
    b1r = b1.reshape(1, _HIDDEN)
    b2c = b2[:_NA].reshape(_NA, 1)
    actions3 = actions_batch.reshape(1, _B, _MAX_T)

    grid_spec = pltpu.PrefetchScalarGridSpec(
        num_scalar_prefetch=1,
        grid=(_B,),
        in_specs=[
            pl.BlockSpec((1, _MAX_T + 1, _S_DIM), lambda i, lens: (i, 0, 0)),
            pl.BlockSpec((1, _B, _MAX_T), lambda i, lens: (0, 0, 0)),
            pl.BlockSpec((_S_DIM, _HIDDEN), lambda i, lens: (0, 0)),
            pl.BlockSpec((1, _HIDDEN), lambda i, lens: (0, 0)),
            pl.BlockSpec((_NA, _HIDDEN), lambda i, lens: (0, 0)),
            pl.BlockSpec((_NA, 1), lambda i, lens: (0, 0)),
        ],
        out_specs=pl.BlockSpec(memory_space=pltpu.SMEM),
    )

    total = pl.pallas_call(
        _traj_kernel,
        grid_spec=grid_spec,
        out_shape=jax.ShapeDtypeStruct((1, 1), jnp.float32),
        compiler_params=pltpu.CompilerParams(
            dimension_semantics=("arbitrary",)),
    )(lengths, s_i_batch, actions3, W1, b1r, w2t, b2c)
    return -total[0, 0]


# speedup vs baseline: 1.7700x; 1.7700x over previous
import jax
import jax.numpy as jnp
from jax.experimental import pallas as pl
from jax.experimental.pallas import tpu as pltpu

_B = 16
_MAX_T = 4096
_S_DIM = 64
_HIDDEN = 128
_NA = 4
_CT = 1024
_NC = _MAX_T // _CT


def _traj_kernel(lens_ref, st_ref, a_ref, w1t_ref, b1_ref, w2t_ref, b2_ref,
                 out_ref):
    i = pl.program_id(0)
    length = lens_ref[i]

    @pl.when(i == 0)
    def _init():
        out_ref[0, 0] = 0.0

    acts = a_ref[0, pl.ds(i, 1), :]

    for c in range(_NC):
        @pl.when(c * _CT < length)
        def _chunk(c=c):
            xt = st_ref[0, :, c * _CT:(c + 1) * _CT]
            ht = jnp.tanh(
                jax.lax.dot_general(w1t_ref[...], xt, (((1,), (0,)), ((), ())),
                                    preferred_element_type=jnp.float32)
                + b1_ref[...])
            zt = jax.lax.dot_general(w2t_ref[...], ht, (((1,), (0,)), ((), ())),
                                     preferred_element_type=jnp.float32)
            zt = zt + b2_ref[...]
            m = jnp.max(zt, axis=0, keepdims=True)
            lse = m + jnp.log(jnp.sum(jnp.exp(zt - m), axis=0, keepdims=True))
            logp = zt - lse
            a_c = acts[:, c * _CT:(c + 1) * _CT]
            onehot = (a_c ==
                      jax.lax.broadcasted_iota(jnp.int32, (_NA, _CT), 0))
            t = c * _CT + jax.lax.broadcasted_iota(jnp.int32, (1, _CT), 1)
            sel = jnp.where(onehot & (t < length), logp, 0.0)
            out_ref[0, 0] += jnp.sum(sel)


def kernel(s_i_batch, actions_batch, lengths, W1, b1, W2, b2):
    st = jnp.swapaxes(s_i_batch, 1, 2)
    w1t = W1.T
    w2t = W2[:, :_NA].T
    b1c = b1.reshape(_HIDDEN, 1)
    b2c = b2[:_NA].reshape(_NA, 1)
    actions3 = actions_batch.reshape(1, _B, _MAX_T)

    grid_spec = pltpu.PrefetchScalarGridSpec(
        num_scalar_prefetch=1,
        grid=(_B,),
        in_specs=[
            pl.BlockSpec((1, _S_DIM, _MAX_T + 1), lambda i, lens: (i, 0, 0)),
            pl.BlockSpec((1, _B, _MAX_T), lambda i, lens: (0, 0, 0)),
            pl.BlockSpec((_HIDDEN, _S_DIM), lambda i, lens: (0, 0)),
            pl.BlockSpec((_HIDDEN, 1), lambda i, lens: (0, 0)),
            pl.BlockSpec((_NA, _HIDDEN), lambda i, lens: (0, 0)),
            pl.BlockSpec((_NA, 1), lambda i, lens: (0, 0)),
        ],
        out_specs=pl.BlockSpec(memory_space=pltpu.SMEM),
    )

    total = pl.pallas_call(
        _traj_kernel,
        grid_spec=grid_spec,
        out_shape=jax.ShapeDtypeStruct((1, 1), jnp.float32),
        compiler_params=pltpu.CompilerParams(
            dimension_semantics=("arbitrary",)),
    )(lengths, st, actions3, w1t, b1c, w2t, b2c)
    return -total[0, 0]


# speedup vs baseline: 2.3746x; 1.3415x over previous
import jax
import jax.numpy as jnp
from jax.experimental import pallas as pl
from jax.experimental.pallas import tpu as pltpu

_B = 16
_MAX_T = 4096
_S_DIM = 64
_HIDDEN = 128
_NA = 4
_CT = 1024
_NC = _MAX_T // _CT


def _traj_kernel(lens_ref, st_ref, a_ref, w1t_ref, b1_ref, w2t_ref, b2_ref,
                 out_ref):
    i = pl.program_id(0)
    length = lens_ref[i]

    @pl.when(i == 0)
    def _init():
        out_ref[0, 0] = 0.0

    acts = a_ref[0, pl.ds(i, 1), :]

    xt = st_ref[0, :, : _MAX_T]
    ht = jnp.tanh(
        jax.lax.dot_general(w1t_ref[...], xt.astype(jnp.bfloat16),
                            (((1,), (0,)), ((), ())),
                            preferred_element_type=jnp.float32)
        + b1_ref[...])
    zt = jax.lax.dot_general(w2t_ref[...], ht.astype(jnp.bfloat16),
                             (((1,), (0,)), ((), ())),
                             preferred_element_type=jnp.float32)
    zt = zt + b2_ref[...]
    m = jnp.max(zt, axis=0, keepdims=True)
    lse = m + jnp.log(jnp.sum(jnp.exp(zt - m), axis=0, keepdims=True))
    logp = zt - lse
    onehot = (acts ==
              jax.lax.broadcasted_iota(jnp.int32, (_NA, _MAX_T), 0))
    t = jax.lax.broadcasted_iota(jnp.int32, (1, _MAX_T), 1)
    sel = jnp.where(onehot & (t < length), logp, 0.0)
    out_ref[0, 0] += jnp.sum(sel)


def kernel(s_i_batch, actions_batch, lengths, W1, b1, W2, b2):
    st = jnp.swapaxes(s_i_batch, 1, 2)
    w1t = W1.T.astype(jnp.bfloat16)
    w2t = W2[:, :_NA].T.astype(jnp.bfloat16)
    b1c = b1.reshape(_HIDDEN, 1)
    b2c = b2[:_NA].reshape(_NA, 1)
    actions3 = actions_batch.reshape(1, _B, _MAX_T)

    grid_spec = pltpu.PrefetchScalarGridSpec(
        num_scalar_prefetch=1,
        grid=(_B,),
        in_specs=[
            pl.BlockSpec((1, _S_DIM, _MAX_T + 1), lambda i, lens: (i, 0, 0)),
            pl.BlockSpec((1, _B, _MAX_T), lambda i, lens: (0, 0, 0)),
            pl.BlockSpec((_HIDDEN, _S_DIM), lambda i, lens: (0, 0)),
            pl.BlockSpec((_HIDDEN, 1), lambda i, lens: (0, 0)),
            pl.BlockSpec((_NA, _HIDDEN), lambda i, lens: (0, 0)),
            pl.BlockSpec((_NA, 1), lambda i, lens: (0, 0)),
        ],
        out_specs=pl.BlockSpec(memory_space=pltpu.SMEM),
    )

    total = pl.pallas_call(
        _traj_kernel,
        grid_spec=grid_spec,
        out_shape=jax.ShapeDtypeStruct((1, 1), jnp.float32),
        compiler_params=pltpu.CompilerParams(
            dimension_semantics=("arbitrary",)),
    )(lengths, st, actions3, w1t, b1c, w2t, b2c)
    return -total[0, 0]
